# baseline (speedup 1.0000x reference)
import jax
import jax.numpy as jnp
from jax import lax
from jax.experimental import pallas as pl
from jax.experimental.pallas import tpu as pltpu
from jax.experimental.pallas import tpu_sc as plsc

_B = 16
_B_TEC = 10
_NC = 2
_NS = 16
_NW = _NC * _NS


def _tec_body(states_hbm, out_hbm):
    _, S, D = states_hbm.shape
    rows_per_w = S // _NW

    def inner(rows_v, sem_in, sem_out):
        wid = lax.axis_index("c") * _NS + lax.axis_index("s")
        base = wid * rows_per_w
        stage = pltpu.make_async_copy(
            states_hbm.at[0, pl.ds(base, rows_per_w)], rows_v, sem_in
        )
        stage.start()
        stage.wait()
        writes = [
            pltpu.make_async_copy(
                rows_v, out_hbm.at[b, pl.ds(base, rows_per_w)], sem_out
            )
            for b in range(_B_TEC)
        ]
        for c in writes:
            c.start()
        for c in writes:
            c.wait()

    pl.run_scoped(
        inner,
        pltpu.MemorySpace.VMEM((rows_per_w, D), states_hbm.dtype),
        pltpu.SemaphoreType.DMA,
        pltpu.SemaphoreType.DMA,
    )


def _scs_body(states_hbm, out_hbm):
    _, S, D = states_hbm.shape
    rows_per_c = S // _NC

    def inner(rows_sh, sem_in, sem_out):
        cid = lax.axis_index("c")
        base = cid * rows_per_c
        stage = pltpu.make_async_copy(
            states_hbm.at[0, pl.ds(base, rows_per_c)], rows_sh, sem_in
        )
        stage.start()
        stage.wait()
        writes = [
            pltpu.make_async_copy(
                rows_sh, out_hbm.at[b, pl.ds(base, rows_per_c)], sem_out
            )
            for b in range(_B_TEC, _B)
        ]
        for c in writes:
            c.start()
        for c in writes:
            c.wait()

    pl.run_scoped(
        inner,
        pltpu.MemorySpace.VMEM_SHARED((rows_per_c, D), states_hbm.dtype),
        pltpu.SemaphoreType.DMA,
        pltpu.SemaphoreType.DMA,
    )


def kernel(states, batch_size):
    del batch_size
    _, S, D = states.shape
    call = pl.kernel(
        [_scs_body, _tec_body],
        out_type=jax.ShapeDtypeStruct((_B, S, D), states.dtype),
        mesh=[
            plsc.ScalarSubcoreMesh(axis_name="c", num_cores=_NC),
            plsc.VectorSubcoreMesh(core_axis_name="c", subcore_axis_name="s"),
        ],
    )
    return call(states)


# speedup vs baseline: 1.0193x; 1.0193x over previous
import jax
import jax.numpy as jnp
from jax import lax
from jax.experimental import pallas as pl
from jax.experimental.pallas import tpu as pltpu
from jax.experimental.pallas import tpu_sc as plsc

_B = 16
_NC = 2
_NS = 16
_NW = _NC * _NS


def _sc_body(states_hbm, out_hbm, rows_v, sem_in, sem_out):
    rows_per_w = rows_v.shape[0]
    half = rows_per_w // 2
    wid = lax.axis_index("c") * _NS + lax.axis_index("s")
    base = wid * rows_per_w
    stages = [
        pltpu.make_async_copy(
            states_hbm.at[0, pl.ds(base + h * half, half)],
            rows_v.at[pl.ds(h * half, half)],
            sem_in,
        )
        for h in range(2)
    ]
    for st in stages:
        st.start()
    writes = []
    for h in range(2):
        stages[h].wait()
        for b in range(_B):
            c = pltpu.make_async_copy(
                rows_v.at[pl.ds(h * half, half)],
                out_hbm.at[b, pl.ds(base + h * half, half)],
                sem_out,
            )
            c.start()
            writes.append(c)
    for c in writes:
        c.wait()


def kernel(states, batch_size):
    del batch_size
    _, S, D = states.shape
    rows_per_w = S // _NW
    sc_call = pl.kernel(
        _sc_body,
        out_type=jax.ShapeDtypeStruct((_B, S, D), states.dtype),
        mesh=plsc.VectorSubcoreMesh(core_axis_name="c", subcore_axis_name="s"),
        scratch_types=[
            pltpu.MemorySpace.VMEM((rows_per_w, D), states.dtype),
            pltpu.SemaphoreType.DMA,
            pltpu.SemaphoreType.DMA,
        ],
    )
    return sc_call(states)
